# baseline (speedup 1.0000x reference)
import functools

import jax
import jax.numpy as jnp
from jax import lax
from jax.experimental import pallas as pl
from jax.experimental.pallas import tpu as pltpu
from jax.experimental.pallas import tpu_sc as plsc

_B, _C, _HW = 32, 384, 1024
_CPC = 16
_CH = _CPC * _HW
_N = _C // _CPC
_K = 4
_L = 2


def _sc_body(x_hbm, mid_hbm, out_hbm, b0, b1, b2, b3, mask_v,
             is0, is1, is2, is3, os0, os1, os2, os3):
    bufs = (b0, b1, b2, b3)
    isems = (is0, is1, is2, is3)
    osems = (os0, os1, os2, os3)

    wid = lax.axis_index("s") * 2 + lax.axis_index("c")

    pltpu.sync_copy(mid_hbm, mask_v)
    wid_vec = jnp.zeros((16,), jnp.int32) + wid
    mid_vec = plsc.load_gather(mask_v, [wid_vec])
    idx_vec = lax.iota(jnp.int32, 16) * _HW + mid_vec
    zeros = jnp.zeros((16,), jnp.float32)

    def start_in(k):
        s = k % _K
        return pltpu.make_async_copy(
            x_hbm.at[wid, pl.ds(k * _CH, _CH)], bufs[s], isems[s]).start()

    def wait_in(k):
        s = k % _K
        pltpu.make_async_copy(
            x_hbm.at[wid, pl.ds(k * _CH, _CH)], bufs[s], isems[s]).wait()

    def start_out(k):
        s = k % _K
        return pltpu.make_async_copy(
            bufs[s], out_hbm.at[wid, pl.ds(k * _CH, _CH)], osems[s]).start()

    def wait_out(k):
        s = k % _K
        pltpu.make_async_copy(
            bufs[s], out_hbm.at[wid, pl.ds(k * _CH, _CH)], osems[s]).wait()

    for k in range(_L):
        start_in(k)
    for k in range(_N):
        nxt = k + _L
        if nxt < _N:
            if nxt >= _K:
                wait_out(nxt - _K)
            start_in(nxt)
        wait_in(k)
        plsc.store_scatter(bufs[k % _K], [idx_vec], zeros)
        start_out(k)
    for k in range(_N - _K, _N):
        wait_out(k)


def kernel(fea, mask_id):
    b, c, h, w = fea.shape
    x = fea.reshape(b, c * h * w)
    mesh = plsc.VectorSubcoreMesh(core_axis_name="c", subcore_axis_name="s")
    run = functools.partial(
        pl.kernel,
        mesh=mesh,
        compiler_params=pltpu.CompilerParams(needs_layout_passes=False),
        out_type=jax.ShapeDtypeStruct((b, c * h * w), jnp.float32),
        scratch_types=(
            [pltpu.VMEM((_CH,), jnp.float32) for _ in range(_K)]
            + [pltpu.VMEM((_B,), jnp.int32)]
            + [pltpu.SemaphoreType.DMA for _ in range(2 * _K)]
        ),
    )(_sc_body)
    out = run(x, mask_id)
    return out.reshape(b, c, h, w)


# speedup vs baseline: 4.1133x; 4.1133x over previous
import jax
import jax.numpy as jnp
from jax.experimental import pallas as pl
from jax.experimental.pallas import tpu as pltpu

_K = 8


def _body(x_ref, mid_ref, o_ref, ibuf, obuf, isem, osem):
    n = pl.num_programs(0)
    i = pl.program_id(0)
    slot = jax.lax.rem(i, _K)
    hw = x_ref.shape[-1]

    @pl.when(i == 0)
    def _prologue():
        for k in range(_K):
            pltpu.make_async_copy(x_ref.at[k], ibuf.at[k], isem.at[k]).start()

    pltpu.make_async_copy(x_ref.at[i], ibuf.at[slot], isem.at[slot]).wait()

    @pl.when(i >= _K)
    def _wait_out():
        pltpu.make_async_copy(obuf.at[slot], o_ref.at[i - _K], osem.at[slot]).wait()

    mid = mid_ref[i]
    pos = jax.lax.broadcasted_iota(jnp.int32, (1, hw), 1)
    obuf[slot] = jnp.where(pos == mid, 0.0, ibuf[slot])

    pltpu.make_async_copy(obuf.at[slot], o_ref.at[i], osem.at[slot]).start()

    @pl.when(i + _K < n)
    def _next_in():
        pltpu.make_async_copy(x_ref.at[i + _K], ibuf.at[slot], isem.at[slot]).start()

    @pl.when(i == n - 1)
    def _drain():
        for k in range(_K):
            j = i - (_K - 1) + k
            sl = jax.lax.rem(j, _K)
            pltpu.make_async_copy(obuf.at[sl], o_ref.at[j], osem.at[sl]).wait()


def kernel(fea, mask_id):
    b, c, h, w = fea.shape
    hw = h * w
    x = fea.reshape(b, c, hw)
    out = pl.pallas_call(
        _body,
        grid=(b,),
        in_specs=[
            pl.BlockSpec(memory_space=pl.ANY),
            pl.BlockSpec(memory_space=pltpu.SMEM),
        ],
        out_specs=pl.BlockSpec(memory_space=pl.ANY),
        out_shape=jax.ShapeDtypeStruct((b, c, hw), jnp.float32),
        scratch_shapes=[
            pltpu.VMEM((_K, c, hw), jnp.float32),
            pltpu.VMEM((_K, c, hw), jnp.float32),
            pltpu.SemaphoreType.DMA((_K,)),
            pltpu.SemaphoreType.DMA((_K,)),
        ],
    )(x, mask_id)
    return out.reshape(b, c, h, w)
